# baseline (speedup 1.0000x reference)
import functools

import jax
import jax.numpy as jnp
from jax import lax
from jax.experimental import pallas as pl
from jax.experimental.pallas import tpu as pltpu
from jax.experimental.pallas import tpu_sc as plsc

_NODES = 10000
_EDGES = 320000
_H = 128
_NC, _NS = 2, 16
_NW = _NC * _NS
_EPW = _EDGES // _NW

_CH = 128
_NFULL = _EPW // _CH
_TAIL = _EPW - _NFULL * _CH
_NT = _NFULL + 1
_RPS = 624

_MESH = dict(core_axis_name="c", subcore_axis_name="s")


def _chunk_n(t):
  return _TAIL if t == _NFULL else _CH


def _vec_rows(buf_a, buf_b, nrows):
  def row(r, _):
    for j in range(_H // 16):
      sl = pl.ds(j * 16, 16)
      if buf_b is None:
        buf_a[r, sl] = jnp.maximum(buf_a[r, sl], 0.0)
      else:
        buf_a[r, sl] = buf_a[r, sl] + buf_b[r, sl]
    return 0
  lax.fori_loop(0, nrows, row, 0)


def _mv_partials(E, dest):
  @functools.partial(
      pl.kernel,
      out_type=jax.ShapeDtypeStruct((_NC, _NODES, _H), jnp.float32),
      mesh=plsc.VectorSubcoreMesh(**_MESH),
      scratch_types=(
          [pltpu.VMEM((_CH, _H), jnp.float32) for _ in range(3)]
          + [pltpu.VMEM((1, _CH), jnp.int32) for _ in range(3)]
          + [pltpu.VMEM((1, _TAIL), jnp.int32),
             pltpu.VMEM_SHARED((_NODES, _H), jnp.float32)]
          + [pltpu.SemaphoreType.DMA for _ in range(9)]
      ),
  )
  def k(e_hbm, d_hbm, p_hbm, eb0, eb1, eb2, ib0, ib1, ib2, ibt, table,
        se0, se1, se2, si0, si1, si2, ss0, ss1, ss2):
    c = lax.axis_index("c")
    s = lax.axis_index("s")
    w = s * _NC + c
    ebs, ibs = [eb0, eb1, eb2], [ib0, ib1, ib2]
    semE, semI, semS = [se0, se1, se2], [si0, si1, si2], [ss0, ss1, ss2]

    def zrow(r, _):
      for j in range(_H // 16):
        eb0[r, pl.ds(j * 16, 16)] = jnp.zeros((16,), jnp.float32)
      return 0

    lax.fori_loop(0, _CH, zrow, 0)
    row0 = s * _RPS
    for off in range(0, _RPS, _CH):
      n = min(_CH, _RPS - off)
      pltpu.sync_copy(eb0.at[pl.ds(0, n)], table.at[pl.ds(row0 + off, n)])

    @pl.when(s == _NS - 1)
    def _():
      pltpu.sync_copy(eb0.at[pl.ds(0, _NODES - _NS * _RPS)],
                      table.at[pl.ds(_NS * _RPS, _NODES - _NS * _RPS)])

    plsc.subcore_barrier()

    base = w * _EPW
    dE, dI, dS = [None] * _NT, [None] * _NT, [None] * _NT

    def fire_load(t):
      p = t % 3
      n = _chunk_n(t)
      ib_dst = ibt.at[0] if t == _NFULL else ibs[p].at[0]
      eb_dst = ebs[p].at[pl.ds(0, n)] if n != _CH else ebs[p]
      dI[t] = pltpu.async_copy(d_hbm.at[pl.ds(base + t * _CH, n)], ib_dst,
                               semI[p])
      dE[t] = pltpu.async_copy(e_hbm.at[pl.ds(base + t * _CH, n)], eb_dst,
                               semE[p])

    fire_load(0)
    for t in range(_NT):
      p = t % 3
      n = _chunk_n(t)
      if t >= 2:
        dS[t - 2].wait()
      if t + 1 < _NT:
        fire_load(t + 1)
      dE[t].wait()
      dI[t].wait()
      _vec_rows(ebs[p], None, n)
      idxref = ibt.at[0] if t == _NFULL else ibs[p].at[0]
      src = ebs[p].at[pl.ds(0, n)] if n != _CH else ebs[p]
      dS[t] = pltpu.async_copy(src, table.at[idxref], semS[p], add=True)
    dS[_NT - 2].wait()
    dS[_NT - 1].wait()

    plsc.subcore_barrier()
    pltpu.sync_copy(table.at[pl.ds(row0, _RPS)],
                    p_hbm.at[c, pl.ds(row0, _RPS)])

    @pl.when(s == _NS - 1)
    def _():
      pltpu.sync_copy(table.at[pl.ds(_NS * _RPS, _NODES - _NS * _RPS)],
                      p_hbm.at[c, pl.ds(_NS * _RPS, _NODES - _NS * _RPS)])

  return k(E, dest)


def _combine_matmul(P, W, b2):
  blk = 5000

  def body(p_ref, w_ref, b_ref, o_ref):
    m = p_ref[0] + p_ref[1]
    o_ref[...] = lax.dot_general(
        m, w_ref[...], (((1,), (1,)), ((), ())),
        preferred_element_type=jnp.float32) + b_ref[...]

  return pl.pallas_call(
      body,
      grid=(_NODES // blk,),
      in_specs=[pl.BlockSpec((2, blk, _H), lambda i: (0, i, 0)),
                pl.BlockSpec((_H, _H), lambda i: (0, 0)),
                pl.BlockSpec((1, _H), lambda i: (0, 0))],
      out_specs=pl.BlockSpec((blk, _H), lambda i: (i, 0)),
      out_shape=jax.ShapeDtypeStruct((_NODES, _H), jnp.float32),
  )(P, W, b2)


def _neg_relu_matmul(E, W):
  blk = 5000

  def body(e_ref, w_ref, o_ref):
    h = jnp.maximum(e_ref[...], 0.0)
    o_ref[...] = -lax.dot_general(h, w_ref[...], (((1,), (1,)), ((), ())),
                                  preferred_element_type=jnp.float32)

  return pl.pallas_call(
      body,
      grid=(_EDGES // blk,),
      in_specs=[pl.BlockSpec((blk, _H), lambda i: (i, 0)),
                pl.BlockSpec((_H, _H), lambda i: (0, 0))],
      out_specs=pl.BlockSpec((blk, _H), lambda i: (i, 0)),
      out_shape=jax.ShapeDtypeStruct((_EDGES, _H), jnp.float32),
  )(E, W)


def _split_edges(ei):
  def body(e_ref, s_ref, d_ref):
    s_ref[...] = e_ref[0]
    d_ref[...] = e_ref[1]

  return pl.pallas_call(
      body,
      out_shape=[jax.ShapeDtypeStruct((_EDGES,), jnp.int32),
                 jax.ShapeDtypeStruct((_EDGES,), jnp.int32)],
  )(ei)


def _gather_combine(A, Bneg, src, rev):
  @functools.partial(
      pl.kernel,
      out_type=jax.ShapeDtypeStruct((_EDGES, _H), jnp.float32),
      mesh=plsc.VectorSubcoreMesh(**_MESH),
      scratch_types=(
          [pltpu.VMEM((_CH, _H), jnp.float32) for _ in range(3)]
          + [pltpu.VMEM((_CH,), jnp.int32) for _ in range(6)]
          + [pltpu.VMEM_SHARED((_NODES, _H), jnp.float32)]
          + [pltpu.SemaphoreType.DMA for _ in range(12)]
      ),
  )
  def k(a_hbm, b_hbm, s_hbm, r_hbm, o_hbm,
        bb0, bb1, bb2, is0, is1, is2, ir0, ir1, ir2, atab,
        gi0, gi1, gi2, gg0, gg1, gg2, ga0, ga1, ga2, gt0, gt1, gt2):
    c = lax.axis_index("c")
    s = lax.axis_index("s")
    w = s * _NC + c
    bbs = [bb0, bb1, bb2]
    iss, irs = [is0, is1, is2], [ir0, ir1, ir2]
    semI, semG = [gi0, gi1, gi2], [gg0, gg1, gg2]
    semA, semT = [ga0, ga1, ga2], [gt0, gt1, gt2]

    row0 = s * _RPS
    pltpu.sync_copy(a_hbm.at[pl.ds(row0, _RPS)], atab.at[pl.ds(row0, _RPS)])

    @pl.when(s == _NS - 1)
    def _():
      pltpu.sync_copy(a_hbm.at[pl.ds(_NS * _RPS, _NODES - _NS * _RPS)],
                      atab.at[pl.ds(_NS * _RPS, _NODES - _NS * _RPS)])

    plsc.subcore_barrier()

    base = w * _EPW
    dI, dGb, dGa, dT = ([None] * _NT for _ in range(4))

    def fire_idx(t):
      p = t % 3
      n = _chunk_n(t)
      sl = pl.ds(0, n)
      dI[t] = (
          pltpu.async_copy(s_hbm.at[pl.ds(base + t * _CH, n)],
                           iss[p].at[sl] if n != _CH else iss[p], semI[p]),
          pltpu.async_copy(r_hbm.at[pl.ds(base + t * _CH, n)],
                           irs[p].at[sl] if n != _CH else irs[p], semI[p]),
      )

    def fire_gather(t):
      p = t % 3
      n = _chunk_n(t)
      sl = pl.ds(0, n)
      ri = irs[p].at[sl] if n != _CH else irs[p]
      dGb[t] = pltpu.async_copy(b_hbm.at[ri],
                                bbs[p].at[sl] if n != _CH else bbs[p],
                                semG[p])

    def fire_store(t):
      p = t % 3
      n = _chunk_n(t)
      src_buf = bbs[p].at[pl.ds(0, n)] if n != _CH else bbs[p]
      dT[t] = pltpu.async_copy(src_buf, o_hbm.at[pl.ds(base + t * _CH, n)],
                               semT[p])

    fire_idx(0)
    fire_idx(1)
    dI[0][0].wait()
    dI[0][1].wait()
    fire_gather(0)
    for t in range(_NT):
      p = t % 3
      n = _chunk_n(t)
      sl = pl.ds(0, n)
      if t >= 2:
        dT[t - 2].wait()
      if t + 1 < _NT:
        dI[t + 1][0].wait()
        dI[t + 1][1].wait()
        fire_gather(t + 1)
      if t + 2 < _NT:
        fire_idx(t + 2)
      if t >= 1:
        dGa[t - 1].wait()
        fire_store(t - 1)
      dGb[t].wait()
      si = iss[p].at[sl] if n != _CH else iss[p]
      dGa[t] = pltpu.async_copy(atab.at[si],
                                bbs[p].at[sl] if n != _CH else bbs[p],
                                semA[p], add=True)
    dGa[_NT - 1].wait()
    fire_store(_NT - 1)
    dT[_NT - 2].wait()
    dT[_NT - 1].wait()

  return k(A, Bneg, src, rev)


def kernel(V, E, edge_index, rev_index, W, b):
  src, dest = _split_edges(edge_index)
  P = _mv_partials(E, dest)
  Bneg = _neg_relu_matmul(E, W)
  A = _combine_matmul(P, W, b.reshape(1, _H))
  return _gather_combine(A, Bneg, src, rev_index)


# speedup vs baseline: 1.0025x; 1.0025x over previous
import functools

import jax
import jax.numpy as jnp
from jax import lax
from jax.experimental import pallas as pl
from jax.experimental.pallas import tpu as pltpu
from jax.experimental.pallas import tpu_sc as plsc

_NODES = 10000
_EDGES = 320000
_H = 128
_NC, _NS = 2, 16
_NW = _NC * _NS
_EPW = _EDGES // _NW

_CH = 128
_NFULL = _EPW // _CH
_TAIL = _EPW - _NFULL * _CH
_NT = _NFULL + 1
_RPS = 624

_MESH = dict(core_axis_name="c", subcore_axis_name="s")


def _chunk_n(t):
  return _TAIL if t == _NFULL else _CH


def _vec_rows(buf_a, buf_b, nrows):
  def row(r, _):
    for j in range(_H // 16):
      sl = pl.ds(j * 16, 16)
      if buf_b is None:
        buf_a[r, sl] = jnp.maximum(buf_a[r, sl], 0.0)
      else:
        buf_a[r, sl] = buf_a[r, sl] + buf_b[r, sl]
    return 0
  lax.fori_loop(0, nrows, row, 0)


def _mv_partials(E, dest):
  @functools.partial(
      pl.kernel,
      out_type=jax.ShapeDtypeStruct((_NC, _NODES, _H), jnp.float32),
      mesh=plsc.VectorSubcoreMesh(**_MESH),
      scratch_types=(
          [pltpu.VMEM((_CH, _H), jnp.float32) for _ in range(3)]
          + [pltpu.VMEM((1, _CH), jnp.int32) for _ in range(3)]
          + [pltpu.VMEM((1, _TAIL), jnp.int32),
             pltpu.VMEM_SHARED((_NODES, _H), jnp.float32)]
          + [pltpu.SemaphoreType.DMA for _ in range(9)]
      ),
  )
  def k(e_hbm, d_hbm, p_hbm, eb0, eb1, eb2, ib0, ib1, ib2, ibt, table,
        se0, se1, se2, si0, si1, si2, ss0, ss1, ss2):
    c = lax.axis_index("c")
    s = lax.axis_index("s")
    w = s * _NC + c
    ebs, ibs = [eb0, eb1, eb2], [ib0, ib1, ib2]
    semE, semI, semS = [se0, se1, se2], [si0, si1, si2], [ss0, ss1, ss2]

    def zrow(r, _):
      for j in range(_H // 16):
        eb0[r, pl.ds(j * 16, 16)] = jnp.zeros((16,), jnp.float32)
      return 0

    lax.fori_loop(0, _CH, zrow, 0)
    row0 = s * _RPS
    for off in range(0, _RPS, _CH):
      n = min(_CH, _RPS - off)
      pltpu.sync_copy(eb0.at[pl.ds(0, n)], table.at[pl.ds(row0 + off, n)])

    @pl.when(s == _NS - 1)
    def _():
      pltpu.sync_copy(eb0.at[pl.ds(0, _NODES - _NS * _RPS)],
                      table.at[pl.ds(_NS * _RPS, _NODES - _NS * _RPS)])

    plsc.subcore_barrier()

    base = w * _EPW
    dE, dI, dS = [None] * _NT, [None] * _NT, [None] * _NT

    def fire_load(t):
      p = t % 3
      n = _chunk_n(t)
      ib_dst = ibt.at[0] if t == _NFULL else ibs[p].at[0]
      eb_dst = ebs[p].at[pl.ds(0, n)] if n != _CH else ebs[p]
      dI[t] = pltpu.async_copy(d_hbm.at[pl.ds(base + t * _CH, n)], ib_dst,
                               semI[p])
      dE[t] = pltpu.async_copy(e_hbm.at[pl.ds(base + t * _CH, n)], eb_dst,
                               semE[p])

    fire_load(0)
    for t in range(_NT):
      p = t % 3
      n = _chunk_n(t)
      if t >= 2:
        dS[t - 2].wait()
      if t + 1 < _NT:
        fire_load(t + 1)
      dE[t].wait()
      dI[t].wait()
      _vec_rows(ebs[p], None, n)
      idxref = ibt.at[0] if t == _NFULL else ibs[p].at[0]
      src = ebs[p].at[pl.ds(0, n)] if n != _CH else ebs[p]
      dS[t] = pltpu.async_copy(src, table.at[idxref], semS[p], add=True)
    dS[_NT - 2].wait()
    dS[_NT - 1].wait()

    plsc.subcore_barrier()
    pltpu.sync_copy(table.at[pl.ds(row0, _RPS)],
                    p_hbm.at[c, pl.ds(row0, _RPS)])

    @pl.when(s == _NS - 1)
    def _():
      pltpu.sync_copy(table.at[pl.ds(_NS * _RPS, _NODES - _NS * _RPS)],
                      p_hbm.at[c, pl.ds(_NS * _RPS, _NODES - _NS * _RPS)])

  return k(E, dest)


def _combine_matmul(P, W, b2):
  blk = 5000

  def body(p_ref, w_ref, b_ref, o_ref):
    m = p_ref[0] + p_ref[1]
    o_ref[...] = lax.dot_general(
        m, w_ref[...], (((1,), (1,)), ((), ())),
        preferred_element_type=jnp.float32) + b_ref[...]

  return pl.pallas_call(
      body,
      grid=(_NODES // blk,),
      in_specs=[pl.BlockSpec((2, blk, _H), lambda i: (0, i, 0)),
                pl.BlockSpec((_H, _H), lambda i: (0, 0)),
                pl.BlockSpec((1, _H), lambda i: (0, 0))],
      out_specs=pl.BlockSpec((blk, _H), lambda i: (i, 0)),
      out_shape=jax.ShapeDtypeStruct((_NODES, _H), jnp.float32),
  )(P, W, b2)


def _neg_relu_matmul(E, W):
  blk = 4000

  def body(e_ref, w_ref, o_ref):
    h = jnp.maximum(e_ref[...], 0.0)
    o_ref[...] = -lax.dot_general(h, w_ref[...], (((1,), (1,)), ((), ())),
                                  preferred_element_type=jnp.float32)

  return pl.pallas_call(
      body,
      grid=(_EDGES // blk,),
      in_specs=[pl.BlockSpec((blk, _H), lambda i: (i, 0)),
                pl.BlockSpec((_H, _H), lambda i: (0, 0))],
      out_specs=pl.BlockSpec((blk, _H), lambda i: (i, 0)),
      out_shape=jax.ShapeDtypeStruct((_EDGES, _H), jnp.float32),
  )(E, W)


def _split_edges(ei):
  def body(e_ref, s_ref, d_ref):
    s_ref[...] = e_ref[0]
    d_ref[...] = e_ref[1]

  return pl.pallas_call(
      body,
      out_shape=[jax.ShapeDtypeStruct((_EDGES,), jnp.int32),
                 jax.ShapeDtypeStruct((_EDGES,), jnp.int32)],
  )(ei)


def _gather_combine(A, Bneg, src, rev):
  @functools.partial(
      pl.kernel,
      out_type=jax.ShapeDtypeStruct((_EDGES, _H), jnp.float32),
      mesh=plsc.VectorSubcoreMesh(**_MESH),
      scratch_types=(
          [pltpu.VMEM((_CH, _H), jnp.float32) for _ in range(3)]
          + [pltpu.VMEM((_CH,), jnp.int32) for _ in range(6)]
          + [pltpu.VMEM_SHARED((_NODES, _H), jnp.float32)]
          + [pltpu.SemaphoreType.DMA for _ in range(12)]
      ),
  )
  def k(a_hbm, b_hbm, s_hbm, r_hbm, o_hbm,
        bb0, bb1, bb2, is0, is1, is2, ir0, ir1, ir2, atab,
        gi0, gi1, gi2, gg0, gg1, gg2, ga0, ga1, ga2, gt0, gt1, gt2):
    c = lax.axis_index("c")
    s = lax.axis_index("s")
    w = s * _NC + c
    bbs = [bb0, bb1, bb2]
    iss, irs = [is0, is1, is2], [ir0, ir1, ir2]
    semI, semG = [gi0, gi1, gi2], [gg0, gg1, gg2]
    semA, semT = [ga0, ga1, ga2], [gt0, gt1, gt2]

    row0 = s * _RPS
    pltpu.sync_copy(a_hbm.at[pl.ds(row0, _RPS)], atab.at[pl.ds(row0, _RPS)])

    @pl.when(s == _NS - 1)
    def _():
      pltpu.sync_copy(a_hbm.at[pl.ds(_NS * _RPS, _NODES - _NS * _RPS)],
                      atab.at[pl.ds(_NS * _RPS, _NODES - _NS * _RPS)])

    plsc.subcore_barrier()

    base = w * _EPW
    dI, dGb, dGa, dT = ([None] * _NT for _ in range(4))

    def fire_idx(t):
      p = t % 3
      n = _chunk_n(t)
      sl = pl.ds(0, n)
      dI[t] = (
          pltpu.async_copy(s_hbm.at[pl.ds(base + t * _CH, n)],
                           iss[p].at[sl] if n != _CH else iss[p], semI[p]),
          pltpu.async_copy(r_hbm.at[pl.ds(base + t * _CH, n)],
                           irs[p].at[sl] if n != _CH else irs[p], semI[p]),
      )

    def fire_gather(t):
      p = t % 3
      n = _chunk_n(t)
      sl = pl.ds(0, n)
      ri = irs[p].at[sl] if n != _CH else irs[p]
      dGb[t] = pltpu.async_copy(b_hbm.at[ri],
                                bbs[p].at[sl] if n != _CH else bbs[p],
                                semG[p])

    def fire_store(t):
      p = t % 3
      n = _chunk_n(t)
      src_buf = bbs[p].at[pl.ds(0, n)] if n != _CH else bbs[p]
      dT[t] = pltpu.async_copy(src_buf, o_hbm.at[pl.ds(base + t * _CH, n)],
                               semT[p])

    fire_idx(0)
    fire_idx(1)
    dI[0][0].wait()
    dI[0][1].wait()
    fire_gather(0)
    for t in range(_NT):
      p = t % 3
      n = _chunk_n(t)
      sl = pl.ds(0, n)
      if t >= 2:
        dT[t - 2].wait()
      if t + 1 < _NT:
        dI[t + 1][0].wait()
        dI[t + 1][1].wait()
        fire_gather(t + 1)
      if t + 2 < _NT:
        fire_idx(t + 2)
      if t >= 1:
        dGa[t - 1].wait()
        fire_store(t - 1)
      dGb[t].wait()
      si = iss[p].at[sl] if n != _CH else iss[p]
      dGa[t] = pltpu.async_copy(atab.at[si],
                                bbs[p].at[sl] if n != _CH else bbs[p],
                                semA[p], add=True)
    dGa[_NT - 1].wait()
    fire_store(_NT - 1)
    dT[_NT - 2].wait()
    dT[_NT - 1].wait()

  return k(A, Bneg, src, rev)


def kernel(V, E, edge_index, rev_index, W, b):
  src, dest = _split_edges(edge_index)
  P = _mv_partials(E, dest)
  Bneg = _neg_relu_matmul(E, W)
  A = _combine_matmul(P, W, b.reshape(1, _H))
  return _gather_combine(A, Bneg, src, rev_index)
